# Initial kernel scaffold; baseline (speedup 1.0000x reference)
import jax
import jax.numpy as jnp
from jax.experimental import pallas as pl


def kernel(x, table, W, b):
    raise NotImplementedError("write your pallas kernel here")


# speedup vs baseline: 30.9248x; 30.9248x over previous
import functools

import jax
import jax.numpy as jnp
from jax import lax
from jax.experimental import pallas as pl
from jax.experimental.pallas import tpu as pltpu
from jax.experimental.pallas import tpu_sc as plsc

_VOCAB = 1000000
_DIM = 32
_SEQ = 200
_BATCH = 4096

_NC = 2
_NS = 16
_NW = _NC * _NS
_BPW = _BATCH // _NW
_BLK = 8
_NBLK = _BPW // _BLK
_ROWS = _BLK * _SEQ


@functools.partial(
    pl.kernel,
    out_type=jax.ShapeDtypeStruct((_BATCH,), jnp.float32),
    mesh=plsc.VectorSubcoreMesh(core_axis_name="c", subcore_axis_name="s"),
    compiler_params=pltpu.CompilerParams(use_tc_tiling_on_sc=False),
    scratch_types=[
        pltpu.VMEM((_SEQ * _DIM,), jnp.float32),
        pltpu.VMEM((16,), jnp.float32),
        pltpu.VMEM((_ROWS,), jnp.int32),
        pltpu.VMEM((_ROWS, _DIM), jnp.float32),
        pltpu.VMEM((_BPW,), jnp.float32),
        pltpu.VMEM((_BLK * 32,), jnp.float32),
        pltpu.SemaphoreType.DMA,
    ],
)
def _emb_linear_sc(xT_hbm, table_hbm, w_hbm, bias_hbm, out_hbm,
                   w_v, bias_v, idx_v, rows_v, out_v, pad_v, sem):
    wid = lax.axis_index("s") * _NC + lax.axis_index("c")
    base_b = wid * _BPW

    pltpu.sync_copy(w_hbm, w_v)
    pltpu.sync_copy(bias_hbm, bias_v)
    bias_s = bias_v[...][0]

    lanes = lax.iota(jnp.int32, 16)
    zero16 = jnp.zeros((16,), jnp.float32)
    for bb in range(_BLK):
        pad_v[pl.ds(bb * 32 + 16, 16)] = zero16

    def _hsum(bb, c):
        base = bb * 32
        v = c
        for sh in (8, 4, 2, 1):
            pad_v[pl.ds(base, 16)] = v
            v = pad_v[pl.ds(base, 16)] + pad_v[pl.ds(base + sh, 16)]
        return v[0]

    def pair(p, carry):
        vec = jnp.full((16,), bias_s, jnp.float32)
        for half in range(2):
            k = 2 * p + half
            pltpu.sync_copy(
                xT_hbm.at[pl.ds(base_b * _SEQ + k * _ROWS, _ROWS)], idx_v)
            pltpu.async_copy(table_hbm.at[idx_v], rows_v, sem).wait()

            def body(s, acc):
                wlo = w_v[pl.ds(s * _DIM, 16)]
                whi = w_v[pl.ds(s * _DIM + 16, 16)]
                out_acc = []
                for bb in range(_BLK):
                    r = bb * _SEQ + s
                    lo = rows_v[r, pl.ds(0, 16)]
                    hi = rows_v[r, pl.ds(16, 16)]
                    out_acc.append(acc[2 * bb] + lo * wlo)
                    out_acc.append(acc[2 * bb + 1] + hi * whi)
                return tuple(out_acc)

            acc = lax.fori_loop(0, _SEQ, body,
                                tuple(zero16 for _ in range(2 * _BLK)))
            for bb in range(_BLK):
                total = _hsum(bb, acc[2 * bb] + acc[2 * bb + 1])
                vec = jnp.where(lanes == half * _BLK + bb, total, vec)
        out_v[pl.ds(p * 16, 16)] = vec
        return carry

    lax.fori_loop(0, _NBLK // 2, pair, 0)
    pltpu.sync_copy(out_v, out_hbm.at[pl.ds(base_b, _BPW)])


@jax.jit
def kernel(x, table, W, b):
    xT = x.T.reshape(-1)
    w_flat = W.reshape(-1)
    b16 = jnp.broadcast_to(b, (16,))
    return _emb_linear_sc(xT, table, w_flat, b16)


# speedup vs baseline: 32.4735x; 1.0501x over previous
import functools

import jax
import jax.numpy as jnp
from jax import lax
from jax.experimental import pallas as pl
from jax.experimental.pallas import tpu as pltpu
from jax.experimental.pallas import tpu_sc as plsc

_VOCAB = 1000000
_DIM = 32
_SEQ = 200
_BATCH = 4096

_NC = 2
_NS = 16
_NW = _NC * _NS
_BPW = _BATCH // _NW
_BLK = 8
_NBLK = _BPW // _BLK
_ROWS = _BLK * _SEQ


@functools.partial(
    pl.kernel,
    out_type=jax.ShapeDtypeStruct((_BATCH,), jnp.float32),
    mesh=plsc.VectorSubcoreMesh(core_axis_name="c", subcore_axis_name="s"),
    compiler_params=pltpu.CompilerParams(use_tc_tiling_on_sc=False),
    scratch_types=[
        pltpu.VMEM((_SEQ * _DIM,), jnp.float32),
        pltpu.VMEM((16,), jnp.float32),
        pltpu.VMEM((_ROWS,), jnp.int32),
        pltpu.VMEM((_ROWS,), jnp.int32),
        pltpu.VMEM((_ROWS, _DIM), jnp.float32),
        pltpu.VMEM((_ROWS, _DIM), jnp.float32),
        pltpu.VMEM((_BPW,), jnp.float32),
        pltpu.VMEM((_BLK * 32,), jnp.float32),
        pltpu.SemaphoreType.DMA,
        pltpu.SemaphoreType.DMA,
    ],
)
def _emb_linear_sc(xT_hbm, table_hbm, w_hbm, bias_hbm, out_hbm,
                   w_v, bias_v, idx_a, idx_b, rows_a, rows_b, out_v, pad_v,
                   sem_a, sem_b):
    wid = lax.axis_index("s") * _NC + lax.axis_index("c")
    base_b = wid * _BPW

    pltpu.sync_copy(w_hbm, w_v)
    pltpu.sync_copy(bias_hbm, bias_v)
    bias_s = bias_v[...][0]

    def _fetch(k, idx_v, rows_v, sem):
        pltpu.sync_copy(
            xT_hbm.at[pl.ds(base_b * _SEQ + k * _ROWS, _ROWS)], idx_v)
        pltpu.async_copy(table_hbm.at[idx_v], rows_v, sem)

    lanes = lax.iota(jnp.int32, 16)
    zero16 = jnp.zeros((16,), jnp.float32)
    for bb in range(_BLK):
        pad_v[pl.ds(bb * 32 + 16, 16)] = zero16

    def _hsum(bb, c):
        base = bb * 32
        v = c
        for sh in (8, 4, 2, 1):
            pad_v[pl.ds(base, 16)] = v
            v = pad_v[pl.ds(base, 16)] + pad_v[pl.ds(base + sh, 16)]
        return v[0]

    def _reduce_block(rows_v, vec, half):
        def body(s, acc):
            wlo = w_v[pl.ds(s * _DIM, 16)]
            whi = w_v[pl.ds(s * _DIM + 16, 16)]
            out_acc = []
            for bb in range(_BLK):
                r = bb * _SEQ + s
                lo = rows_v[r, pl.ds(0, 16)]
                hi = rows_v[r, pl.ds(16, 16)]
                out_acc.append(acc[2 * bb] + lo * wlo)
                out_acc.append(acc[2 * bb + 1] + hi * whi)
            return tuple(out_acc)

        acc = lax.fori_loop(0, _SEQ, body,
                            tuple(zero16 for _ in range(2 * _BLK)))
        for bb in range(_BLK):
            total = _hsum(bb, acc[2 * bb] + acc[2 * bb + 1])
            vec = jnp.where(lanes == half * _BLK + bb, total, vec)
        return vec

    _fetch(0, idx_a, rows_a, sem_a)

    def pair(p, carry):
        vec = jnp.full((16,), bias_s, jnp.float32)
        pltpu.make_async_copy(table_hbm.at[idx_a], rows_a, sem_a).wait()
        _fetch(2 * p + 1, idx_b, rows_b, sem_b)
        vec = _reduce_block(rows_a, vec, 0)
        pltpu.make_async_copy(table_hbm.at[idx_b], rows_b, sem_b).wait()
        _fetch((2 * p + 2) % _NBLK, idx_a, rows_a, sem_a)
        vec = _reduce_block(rows_b, vec, 1)
        out_v[pl.ds(p * 16, 16)] = vec
        return carry

    lax.fori_loop(0, _NBLK // 2, pair, 0)
    pltpu.make_async_copy(table_hbm.at[idx_a], rows_a, sem_a).wait()
    pltpu.sync_copy(out_v, out_hbm.at[pl.ds(base_b, _BPW)])


@jax.jit
def kernel(x, table, W, b):
    xT = x.T.reshape(-1)
    w_flat = W.reshape(-1)
    b16 = jnp.broadcast_to(b, (16,))
    return _emb_linear_sc(xT, table, w_flat, b16)
